# Initial kernel scaffold; baseline (speedup 1.0000x reference)
import jax
import jax.numpy as jnp
from jax.experimental import pallas as pl


def kernel(x, edge_index, batch, edge_attr, W1, b1, gamma, beta, W2, b2):
    raise NotImplementedError("write your pallas kernel here")


# speedup vs baseline: 10.2530x; 10.2530x over previous
import functools

import jax
import jax.numpy as jnp
from jax import lax
from jax.experimental import pallas as pl
from jax.experimental.pallas import tpu as pltpu
from jax.experimental.pallas import tpu_sc as plsc


def _rowsum_body(x_ref, o_ref):
    o_ref[...] = jnp.sum(x_ref[...], axis=1)


def _gram_body(a_ref, m_ref, c_ref):
    @pl.when(pl.program_id(0) == 0)
    def _init():
        m_ref[...] = jnp.zeros_like(m_ref)
        c_ref[...] = jnp.zeros_like(c_ref)

    a = a_ref[...]
    m_ref[...] += lax.dot_general(
        a, a, (((0,), (0,)), ((), ())), preferred_element_type=jnp.float32
    )
    c_ref[...] += jnp.sum(a, axis=0, keepdims=True)


def _edge_w_body(n_edges, a_ref, m_ref, c_ref, w1_ref, b1_ref, g_ref, be_ref,
                 w1k_ref, w2k_ref, b2_ref, o_ref):
    m = m_ref[...]
    c = c_ref[...]
    g16 = m[0:16, 0:16]
    cs16 = c[:, 0:16]
    for k in range(1, 8):
        g16 = g16 + m[16 * k:16 * (k + 1), 16 * k:16 * (k + 1)]
        cs16 = cs16 + c[:, 16 * k:16 * (k + 1)]
    inv_e = 1.0 / float(n_edges)
    w1 = w1_ref[...]
    t = jnp.dot(cs16 * inv_e, w1, preferred_element_type=jnp.float32)
    mu = t + b1_ref[...]
    p = jnp.dot(g16 * inv_e, w1, preferred_element_type=jnp.float32)
    q = jnp.sum(w1 * p, axis=0, keepdims=True)
    var = q - t * t
    scale = g_ref[...] * lax.rsqrt(var + 1e-5)
    b1f = (b1_ref[...] - mu) * scale + be_ref[...]
    scale128 = jnp.concatenate([scale] * 8, axis=1)
    b1f128 = jnp.concatenate([b1f] * 8, axis=1)
    h = jnp.dot(a_ref[...], w1k_ref[...] * scale128,
                preferred_element_type=jnp.float32) + b1f128
    h = jnp.maximum(h, 0.0)
    o_ref[...] = jnp.dot(h, w2k_ref[...],
                         preferred_element_type=jnp.float32) + b2_ref[0, 0]


def _fin_body(inv_d, p_ref, o_ref):
    o_ref[...] = jnp.sum(p_ref[...], axis=0) * inv_d


def kernel(x, edge_index, batch, edge_attr, W1, b1, gamma, beta, W2, b2):
    n, d = x.shape
    e, de = edge_attr.shape
    g = 64
    h = W1.shape[1]
    pack = 128 // de
    r = e // pack

    s = pl.pallas_call(
        _rowsum_body,
        grid=(1,),
        in_specs=[pl.BlockSpec((n, d), lambda i: (0, 0))],
        out_specs=pl.BlockSpec((n,), lambda i: (0,)),
        out_shape=jax.ShapeDtypeStruct((n,), jnp.float32),
    )(x)

    ar = edge_attr.reshape(r, pack * de)
    br = min(4000, r)
    m128, cs = pl.pallas_call(
        _gram_body,
        grid=(r // br,),
        in_specs=[pl.BlockSpec((br, pack * de), lambda i: (i, 0))],
        out_specs=[
            pl.BlockSpec((pack * de, pack * de), lambda i: (0, 0)),
            pl.BlockSpec((1, pack * de), lambda i: (0, 0)),
        ],
        out_shape=[
            jax.ShapeDtypeStruct((pack * de, pack * de), jnp.float32),
            jax.ShapeDtypeStruct((1, pack * de), jnp.float32),
        ],
    )(ar)

    eye = jnp.eye(pack, dtype=jnp.float32)
    w1k = jnp.kron(eye, W1)
    w2k = jnp.kron(eye, W2)
    small = lambda a: pl.BlockSpec(a.shape, lambda i: tuple(0 for _ in a.shape))
    b1r = b1.reshape(1, h)
    gr = gamma.reshape(1, h)
    ber = beta.reshape(1, h)
    b2r = b2.reshape(1, 1)
    w_packed = pl.pallas_call(
        functools.partial(_edge_w_body, e),
        grid=(r // br,),
        in_specs=[
            pl.BlockSpec((br, pack * de), lambda i: (i, 0)),
            small(m128), small(cs), small(W1), small(b1r), small(gr),
            small(ber), small(w1k), small(w2k), small(b2r),
        ],
        out_specs=pl.BlockSpec((br, pack), lambda i: (i, 0)),
        out_shape=jax.ShapeDtypeStruct((r, pack), jnp.float32),
    )(ar, m128, cs, W1, b1r, gr, ber, w1k, w2k, b2r)
    w = w_packed.reshape(e)

    nc, ns, lanes = 2, 16, 16
    nw = nc * ns
    epw = e // nw
    src = edge_index[0]
    dst = edge_index[1]
    mesh = plsc.VectorSubcoreMesh(
        core_axis_name="c", subcore_axis_name="s", num_cores=nc)

    @functools.partial(
        pl.kernel,
        out_type=jax.ShapeDtypeStruct((nw, lanes * g), jnp.float32),
        mesh=mesh,
        compiler_params=pltpu.CompilerParams(
            needs_layout_passes=False, use_tc_tiling_on_sc=False),
        scratch_types=[
            pltpu.VMEM((epw,), jnp.float32),
            pltpu.VMEM((epw,), jnp.int32),
            pltpu.VMEM((epw,), jnp.int32),
            pltpu.VMEM((n,), jnp.float32),
            pltpu.VMEM((n,), jnp.int32),
            pltpu.VMEM((lanes * g,), jnp.float32),
        ],
    )
    def _sc_seg(w_hbm, src_hbm, dst_hbm, s_hbm, b_hbm, out_hbm,
                w_v, src_v, dst_v, s_v, b_v, acc_v):
        wid = lax.axis_index("s") * nc + lax.axis_index("c")
        base = wid * epw
        pltpu.sync_copy(s_hbm, s_v)
        pltpu.sync_copy(b_hbm, b_v)
        pltpu.sync_copy(w_hbm.at[pl.ds(base, epw)], w_v)
        pltpu.sync_copy(src_hbm.at[pl.ds(base, epw)], src_v)
        pltpu.sync_copy(dst_hbm.at[pl.ds(base, epw)], dst_v)
        for j in range(g):
            acc_v[pl.ds(j * lanes, lanes)] = jnp.zeros((lanes,), jnp.float32)
        lane = lax.iota(jnp.int32, lanes)

        def body(i, carry):
            off = i * lanes
            srcv = src_v[pl.ds(off, lanes)]
            dstv = dst_v[pl.ds(off, lanes)]
            wv = w_v[pl.ds(off, lanes)]
            sv = plsc.load_gather(s_v, [srcv])
            bg = plsc.load_gather(b_v, [dstv])
            idx = lane * g + bg
            plsc.addupdate_scatter(acc_v, [idx], wv * sv)
            return carry

        lax.fori_loop(0, epw // lanes, body, 0)
        pltpu.sync_copy(acc_v, out_hbm.at[wid])

    partials = _sc_seg(w, src, dst, s, batch)

    p2 = partials.reshape(nw * lanes, g)
    out = pl.pallas_call(
        functools.partial(_fin_body, 1.0 / float(d)),
        grid=(1,),
        in_specs=[pl.BlockSpec((nw * lanes, g), lambda i: (0, 0))],
        out_specs=pl.BlockSpec((g,), lambda i: (0,)),
        out_shape=jax.ShapeDtypeStruct((g,), jnp.float32),
    )(p2)
    return out
